# baseline (speedup 1.0000x reference)
import jax
import jax.numpy as jnp
from jax import lax
from jax.experimental import pallas as pl
from jax.experimental.pallas import tpu as pltpu
from jax.experimental.pallas import tpu_sc as plsc

N = 10000
E = 320000
IN_DIM = 32
HID = 128
N_CLASSES = 6
ASSIGN_DIM = 100
F32 = jnp.float32

NS = 16
LANES = 16
Q = 32
NQ = HID // Q
EC = 80
EPT = E // NS
NCHUNK = EPT // EC
NBUF = 1
NPT = N // NS
NVEC = Q // LANES


def _sc_gate(write_enew):
    mesh = plsc.VectorSubcoreMesh(core_axis_name="c", subcore_axis_name="s",
                                  num_cores=2, num_subcores=NS)
    per_q = 3 if write_enew else 1
    out_type = []
    for _ in range(NQ):
        out_type += [jax.ShapeDtypeStruct((N, 2 * Q), F32)]
        if write_enew:
            out_type += [jax.ShapeDtypeStruct((E, Q), F32),
                         jax.ShapeDtypeStruct((8, Q), F32)]
    scratch = [
        pltpu.VMEM((EC,), jnp.int32),
        pltpu.VMEM((EC,), jnp.int32),
        pltpu.VMEM((EC,), jnp.int32),
        pltpu.VMEM((EC,), jnp.int32),
        pltpu.VMEM((EC, 2 * Q), F32),
        pltpu.VMEM((EC, 2 * Q), F32),
        pltpu.VMEM((EC, Q), F32),
        pltpu.VMEM((EC, Q), F32),
        pltpu.VMEM((EC, Q), F32),
        pltpu.VMEM((EC, Q), F32),
        pltpu.VMEM((EC, Q), F32),
        pltpu.VMEM((EC, Q), F32),
        pltpu.VMEM((EC, 2 * Q), F32),
        pltpu.VMEM((EC, 2 * Q), F32),
        pltpu.VMEM((8, Q), F32),
        pltpu.VMEM((8 * NS, Q), F32),
        pltpu.VMEM((NPT, 2 * Q), F32),
        pltpu.VMEM_SHARED((N, 2 * Q), F32),
        pltpu.VMEM_SHARED((8 * NS, Q), F32),
        pltpu.SemaphoreType.DMA,
        pltpu.SemaphoreType.DMA,
        pltpu.SemaphoreType.DMA,
        pltpu.SemaphoreType.DMA,
    ]

    def body(src_hbm, dst_hbm, db0, db1, db2, db3, e0, e1, e2, e3,
             c0, c1, c2, c3, *rest):
        n_out = per_q * NQ
        outs = rest[:n_out]
        (is0, is1, id0, id1, ga0, ga1, ef0, ef1, cv0, cv1,
         en0, en1, sv0, sv1,
         stats_v, stats_all, zbuf, acc_sh, stats_sh,
         smg0, smg1, sms0, sms1) = rest[n_out:]
        idx_s = (is0, is1)
        idx_d = (id0, id1)
        db = (ga0, ga1)
        ef = (ef0, ef1)
        cv = (cv0, cv1)
        en = (en0, en1)
        sv = (sv0, sv1)
        sem_g = (smg0, smg1)
        sem_s = (sms0, sms1)
        db_ts = (db0, db1, db2, db3)
        e_ts = (e0, e1, e2, e3)
        ce_ts = (c0, c1, c2, c3)
        cid = lax.axis_index("c")
        sid = lax.axis_index("s")
        zero = jnp.zeros((LANES,), F32)
        nslice = pl.ds(sid * NPT, NPT)
        ebase = sid * EPT

        def zrow(i, _):
            for j in range(2 * NVEC):
                zbuf[i, pl.ds(j * LANES, LANES)] = zero
            return 0

        def one_pass(q):
            db_t, e_t, ce_t = db_ts[q], e_ts[q], ce_ts[q]
            if write_enew:
                acc_o, en_o, st_o = outs[per_q * q: per_q * (q + 1)]
            else:
                (acc_o,) = outs[per_q * q: per_q * (q + 1)]
                en_o = st_o = None
            lax.fori_loop(0, NPT, zrow, 0)
            pltpu.sync_copy(zbuf, acc_sh.at[nslice])
            plsc.subcore_barrier()

            def compute(b, cr):
                def row4(i4, c2):
                    out = list(c2)
                    for u in range(4):
                        i = i4 * 4 + u
                        for j in range(NVEC):
                            lsl = pl.ds(j * LANES, LANES)
                            env = (db[b][i, lsl] + ef[b][i, lsl]
                                   + cv[b][i, lsl])
                            s = 1.0 / (1.0 + jnp.exp(-env))
                            sv[b][i, pl.ds(Q + j * LANES, LANES)] = s
                            sv[b][i, lsl] = s * db[b][i, pl.ds(Q + j * LANES,
                                                               LANES)]
                            if write_enew:
                                en[b][i, lsl] = env
                                out[j] = out[j] + env
                                out[NVEC + j] = out[NVEC + j] + env * env
                    return tuple(out)

                return lax.fori_loop(0, EC // 4, row4, cr)

            def step(t, carry):
                di = []
                for b in range(2):
                    esl = pl.ds(ebase + (t * 2 + b) * EC, EC)
                    di.append([
                        pltpu.async_copy(src_hbm.at[esl], idx_s[b],
                                         sem_g[b]),
                        pltpu.async_copy(dst_hbm.at[esl], idx_d[b],
                                         sem_g[b]),
                    ])
                dg = []
                for b in range(2):
                    for d in di[b]:
                        d.wait()
                    esl = pl.ds(ebase + (t * 2 + b) * EC, EC)
                    dg.append([
                        pltpu.async_copy(db_t.at[idx_s[b]], db[b],
                                         sem_g[b]),
                        pltpu.async_copy(e_t.at[idx_d[b]], ef[b],
                                         sem_g[b]),
                        pltpu.async_copy(ce_t.at[esl], cv[b], sem_g[b]),
                    ])
                prev = None
                for b in range(2):
                    for d in dg[b]:
                        d.wait()
                    carry = compute(b, carry)
                    esl = pl.ds(ebase + (t * 2 + b) * EC, EC)
                    ds = []
                    if write_enew:
                        ds.append(pltpu.async_copy(en[b], en_o.at[esl],
                                                   sem_s[b]))
                    pltpu.sync_copy(sv[b], acc_sh.at[idx_d[b]], add=True)
                    if prev is not None:
                        for d in prev:
                            d.wait()
                    prev = ds
                for d in prev:
                    d.wait()
                return carry

            carry = lax.fori_loop(0, NCHUNK // 2, step,
                                  (zero,) * (2 * NVEC))
            if write_enew:
                for j in range(NVEC):
                    lsl = pl.ds(j * LANES, LANES)
                    stats_v[0, lsl] = carry[j]
                    stats_v[1, lsl] = carry[NVEC + j]
                pltpu.sync_copy(stats_v, stats_sh.at[pl.ds(sid * 8, 8)])
            plsc.subcore_barrier()
            pltpu.sync_copy(acc_sh.at[nslice], zbuf)
            pltpu.sync_copy(zbuf, acc_o.at[nslice])
            if write_enew:
                @pl.when(sid == 0)
                def _():
                    pltpu.sync_copy(stats_sh, stats_all)

                    def trow(t, cr):
                        out = list(cr)
                        for j in range(NVEC):
                            lsl = pl.ds(j * LANES, LANES)
                            out[j] = out[j] + stats_all[8 * t, lsl]
                            out[NVEC + j] = (out[NVEC + j]
                                             + stats_all[8 * t + 1, lsl])
                        return tuple(out)

                    tot = lax.fori_loop(0, NS, trow, (zero,) * (2 * NVEC))
                    for j in range(NVEC):
                        lsl = pl.ds(j * LANES, LANES)
                        stats_v[0, lsl] = tot[j]
                        stats_v[1, lsl] = tot[NVEC + j]
                    pltpu.sync_copy(stats_v, st_o)
            plsc.subcore_barrier()

        @pl.when(cid == 0)
        def _():
            one_pass(0)
            one_pass(1)

        @pl.when(cid == 1)
        def _():
            one_pass(2)
            one_pass(3)

    return pl.kernel(body, out_type=tuple(out_type), mesh=mesh,
                     scratch_types=scratch,
                     compiler_params=pltpu.CompilerParams(
                         use_tc_tiling_on_sc=False))


_R1 = 2000
_R2 = 2000


def _k1(first_layer):
    def body(*refs):
        if first_layer:
            (hidx, emb, wa, ba, wb, bb, wd, bd, we, be,
             hv_o, ah_o, *tabs) = refs
            iot = lax.broadcasted_iota(jnp.int32, (_R1, IN_DIM), 1)
            onehot = (iot == hidx[...]).astype(F32)
            hv = jnp.dot(onehot, emb[...], preferred_element_type=F32)
            hv_o[...] = hv
        else:
            (hvr, wa, ba, wb, bb, wd, bd, we, be, ah_o, *tabs) = refs
            hv = hvr[...]
        ah_o[...] = jnp.dot(hv, wa[...], preferred_element_type=F32) + ba[...]
        B = jnp.dot(hv, wb[...], preferred_element_type=F32) + bb[...]
        D = jnp.dot(hv, wd[...], preferred_element_type=F32) + bd[...]
        Ez = jnp.dot(hv, we[...], preferred_element_type=F32) + be[...]
        for q in range(NQ):
            sl = slice(q * Q, (q + 1) * Q)
            tabs[q][...] = jnp.concatenate([D[:, sl], B[:, sl]], axis=1)
            tabs[NQ + q][...] = Ez[:, sl]

    grid = (N // _R1,)
    row = lambda r: (r, 0)
    fix = lambda r: (0, 0)
    w_spec = pl.BlockSpec((HID, HID), fix)
    b_spec = pl.BlockSpec((1, HID), fix)
    full_spec = pl.BlockSpec((_R1, HID), row)
    if first_layer:
        in_specs = [pl.BlockSpec((_R1, 1), row), pl.BlockSpec((IN_DIM, HID), fix)]
    else:
        in_specs = [full_spec]
    in_specs += [w_spec, b_spec] * 4
    out_shape = [jax.ShapeDtypeStruct((N, HID), F32)] * (2 if first_layer else 1)
    out_specs = [full_spec] * (2 if first_layer else 1)
    out_shape += [jax.ShapeDtypeStruct((N, 2 * Q), F32)] * NQ
    out_specs += [pl.BlockSpec((_R1, 2 * Q), row)] * NQ
    out_shape += [jax.ShapeDtypeStruct((N, Q), F32)] * NQ
    out_specs += [pl.BlockSpec((_R1, Q), row)] * NQ
    return pl.pallas_call(body, grid=grid, in_specs=in_specs,
                          out_specs=out_specs, out_shape=out_shape)


def _k3(layer):
    def body(*refs):
        ah, nm_r, dn_r, hin, g, b = refs[:6]
        rest = refs[6:]
        hn = ah[...] + nm_r[...] / (dn_r[...] + 1e-6)
        mu = jnp.mean(hn, axis=0, keepdims=True)
        var = jnp.mean(jnp.square(hn - mu), axis=0, keepdims=True)
        hn = (hn - mu) * lax.rsqrt(var + 1e-5) * g[...] + b[...]
        ho = hin[...] + jnp.maximum(hn, 0.0)
        if layer == 3:
            w1, b1, w2, b2, w3, b3, y_o = rest
            y = jnp.maximum(jnp.dot(ho, w1[...], preferred_element_type=F32) + b1[...], 0.0)
            y = jnp.maximum(jnp.dot(y, w2[...], preferred_element_type=F32) + b2[...], 0.0)
            y_o[...] = jnp.dot(y, w3[...], preferred_element_type=F32) + b3[...]
        elif layer == 1:
            sw, sb, h_o, s_o = rest
            h_o[...] = ho
            z = jnp.dot(ho, sw[...], preferred_element_type=F32) + sb[...]
            z = z - jnp.max(z, axis=-1, keepdims=True)
            ez = jnp.exp(z)
            s_o[...] = ez / jnp.sum(ez, axis=-1, keepdims=True)
        else:
            rest[0][...] = ho

    if layer == 3:
        out_shape = jax.ShapeDtypeStruct((N, N_CLASSES), F32)
    elif layer == 1:
        out_shape = [jax.ShapeDtypeStruct((N, HID), F32),
                     jax.ShapeDtypeStruct((N, ASSIGN_DIM), F32)]
    else:
        out_shape = jax.ShapeDtypeStruct((N, HID), F32)
    return pl.pallas_call(body, out_shape=out_shape)


def _ce0():
    def body(er, ew, eb, cw, cb, *couts):
        u = jnp.dot(ew[...], cw[...], preferred_element_type=F32)
        v = jnp.dot(eb[...], cw[...], preferred_element_type=F32) + cb[...]
        ce = er[...] * u + v
        for q in range(NQ):
            couts[q][...] = ce[:, q * Q:(q + 1) * Q]

    grid = (E // _R2,)
    row = lambda r: (r, 0)
    fix = lambda r: (0, 0)
    return pl.pallas_call(
        body, grid=grid,
        in_specs=[pl.BlockSpec((_R2, 1), row), pl.BlockSpec((1, HID), fix),
                  pl.BlockSpec((1, HID), fix), pl.BlockSpec((HID, HID), fix),
                  pl.BlockSpec((1, HID), fix)],
        out_specs=[pl.BlockSpec((_R2, Q), row)] * NQ,
        out_shape=[jax.ShapeDtypeStruct((E, Q), F32)] * NQ)


def _k2(mode):
    def body(*refs):
        ens = refs[:NQ]
        if mode == 0:
            er, ew, eb, mu, inv, g, b, cw, cb = refs[NQ:NQ + 9]
            rest = refs[NQ + 9:]
            e_in = er[...] * ew[...] + eb[...]
        else:
            evp, mu, inv, g, b, cw, cb = refs[NQ:NQ + 7]
            rest = refs[NQ + 7:]
            e_in = evp[...]
        en = jnp.concatenate([r[...] for r in ens], axis=1)
        en = (en - mu[...]) * inv[...] * g[...] + b[...]
        ev = e_in + jnp.maximum(en, 0.0)
        ce = jnp.dot(ev, cw[...], preferred_element_type=F32) + cb[...]
        if mode < 2:
            ev_o = rest[0]
            couts = rest[1:]
            ev_o[...] = ev
        else:
            couts = rest
        for q in range(NQ):
            couts[q][...] = ce[:, q * Q:(q + 1) * Q]

    grid = (E // _R2,)
    row = lambda r: (r, 0)
    fix = lambda r: (0, 0)
    q_spec = pl.BlockSpec((_R2, Q), row)
    full_spec = pl.BlockSpec((_R2, HID), row)
    p_spec = pl.BlockSpec((1, HID), fix)
    in_specs = [q_spec] * NQ
    if mode == 0:
        in_specs += [pl.BlockSpec((_R2, 1), row), p_spec, p_spec]
    else:
        in_specs += [full_spec]
    in_specs += [p_spec, p_spec, p_spec, p_spec,
                 pl.BlockSpec((HID, HID), fix), p_spec]
    out_shape = []
    out_specs = []
    if mode < 2:
        out_shape.append(jax.ShapeDtypeStruct((E, HID), F32))
        out_specs.append(full_spec)
    out_shape += [jax.ShapeDtypeStruct((E, Q), F32)] * NQ
    out_specs += [q_spec] * NQ
    return pl.pallas_call(body, grid=grid, in_specs=in_specs,
                          out_specs=out_specs, out_shape=out_shape)


def kernel(h, e, edge_index, params):
    src = edge_index[0].astype(jnp.int32)
    dst = edge_index[1].astype(jnp.int32)
    p = params
    layers = p['layers']
    r2 = lambda x: x.reshape(1, -1)
    h2 = h.astype(jnp.int32).reshape(N, 1)
    e = e.astype(F32)
    ew, eb = r2(p['emb_e_w']), r2(p['emb_e_b'])

    sc_w = _sc_gate(True)
    sc_n = _sc_gate(False)

    ces = _ce0()(e, ew, eb, layers[0]['C_w'], r2(layers[0]['C_b']))
    hv = None
    ev_prev = None
    s_out = None
    y_out = None
    for l in range(4):
        lp = layers[l]
        proj_args = (lp['A_w'], r2(lp['A_b']), lp['B_w'], r2(lp['B_b']),
                     lp['D_w'], r2(lp['D_b']), lp['E_w'], r2(lp['E_b']))
        if l == 0:
            hv, ah, *tabs = _k1(True)(h2, p['emb_h'], *proj_args)
        else:
            ah, *tabs = _k1(False)(hv, *proj_args)
        hin = hv
        if l < 3:
            sc_outs = sc_w(src, dst, *tabs, *ces)
            accs = sc_outs[0::3]
            ens = sc_outs[1::3]
            sts = sc_outs[2::3]
        else:
            accs = sc_n(src, dst, *tabs, *ces)
        nums = [a[:, :Q] for a in accs]
        dens = [a[:, Q:] for a in accs]
        g, b = r2(lp['bn_h_g']), r2(lp['bn_h_b'])
        nm = jnp.concatenate(nums, axis=1)
        dn = jnp.concatenate(dens, axis=1)
        if l == 3:
            mlp = p['mlp']
            y_out = _k3(3)(ah, nm, dn, hin, g, b,
                           mlp[0]['w'], r2(mlp[0]['b']),
                           mlp[1]['w'], r2(mlp[1]['b']),
                           mlp[2]['w'], r2(mlp[2]['b']))
        elif l == 1:
            hv, s_out = _k3(1)(ah, nm, dn, hin, g, b,
                               lp['S_w'], r2(lp['S_b']))
        else:
            hv = _k3(0)(ah, nm, dn, hin, g, b)
        if l < 3:
            ssum = jnp.concatenate([s[0:1, :] for s in sts], axis=1)
            ssq = jnp.concatenate([s[1:2, :] for s in sts], axis=1)
            mu = ssum / E
            inv = lax.rsqrt(jnp.maximum(ssq / E - mu * mu, 0.0) + 1e-5)
            ge, be_ = r2(lp['bn_e_g']), r2(lp['bn_e_b'])
            cn = layers[l + 1]
            cw, cb = cn['C_w'], r2(cn['C_b'])
            if l == 0:
                ev_prev, *ces = _k2(0)(*ens, e, ew, eb, mu, inv, ge, be_,
                                       cw, cb)
            elif l == 1:
                ev_prev, *ces = _k2(1)(*ens, ev_prev, mu, inv, ge, be_,
                                       cw, cb)
            else:
                ces = _k2(2)(*ens, ev_prev, mu, inv, ge, be_, cw, cb)
    return (y_out, s_out)


# speedup vs baseline: 1.3300x; 1.3300x over previous
import jax
import jax.numpy as jnp
from jax import lax
from jax.experimental import pallas as pl
from jax.experimental.pallas import tpu as pltpu
from jax.experimental.pallas import tpu_sc as plsc

N = 10000
E = 320000
IN_DIM = 32
HID = 128
N_CLASSES = 6
ASSIGN_DIM = 100
F32 = jnp.float32

NS = 16
LANES = 16
Q = 32
NQ = HID // Q
EC = 80
EPT = E // NS
NCHUNK = EPT // EC
NBUF = 1
NPT = N // NS
NVEC = Q // LANES


def _sc_gate(write_enew):
    mesh = plsc.VectorSubcoreMesh(core_axis_name="c", subcore_axis_name="s",
                                  num_cores=2, num_subcores=NS)
    out_type = [jax.ShapeDtypeStruct((N, 2 * Q), F32)] * NQ
    if write_enew:
        out_type += [jax.ShapeDtypeStruct((E, HID), F32)]
        out_type += [jax.ShapeDtypeStruct((8, Q), F32)] * NQ
    scratch = [
        pltpu.VMEM((EC,), jnp.int32),
        pltpu.VMEM((EC,), jnp.int32),
        pltpu.VMEM((EC,), jnp.int32),
        pltpu.VMEM((EC,), jnp.int32),
        pltpu.VMEM((EC, 2 * Q), F32),
        pltpu.VMEM((EC, 2 * Q), F32),
        pltpu.VMEM((EC, Q), F32),
        pltpu.VMEM((EC, Q), F32),
        pltpu.VMEM((EC, Q), F32),
        pltpu.VMEM((EC, Q), F32),
        pltpu.VMEM((EC, Q), F32),
        pltpu.VMEM((EC, Q), F32),
        pltpu.VMEM((EC, 2 * Q), F32),
        pltpu.VMEM((EC, 2 * Q), F32),
        pltpu.VMEM((8, Q), F32),
        pltpu.VMEM((8 * NS, Q), F32),
        pltpu.VMEM((NPT, 2 * Q), F32),
        pltpu.VMEM_SHARED((N, 2 * Q), F32),
        pltpu.VMEM_SHARED((8 * NS, Q), F32),
        pltpu.SemaphoreType.DMA,
        pltpu.SemaphoreType.DMA,
        pltpu.SemaphoreType.DMA,
        pltpu.SemaphoreType.DMA,
    ]

    def body(src_hbm, dst_hbm, db0, db1, db2, db3, e0, e1, e2, e3,
             ce_hbm, *rest):
        n_out = (2 * NQ + 1) if write_enew else NQ
        outs = rest[:n_out]
        (is0, is1, id0, id1, ga0, ga1, ef0, ef1, cv0, cv1,
         en0, en1, sv0, sv1,
         stats_v, stats_all, zbuf, acc_sh, stats_sh,
         smg0, smg1, sms0, sms1) = rest[n_out:]
        idx_s = (is0, is1)
        idx_d = (id0, id1)
        db = (ga0, ga1)
        ef = (ef0, ef1)
        cv = (cv0, cv1)
        en = (en0, en1)
        sv = (sv0, sv1)
        sem_g = (smg0, smg1)
        sem_s = (sms0, sms1)
        db_ts = (db0, db1, db2, db3)
        e_ts = (e0, e1, e2, e3)
        cid = lax.axis_index("c")
        sid = lax.axis_index("s")
        zero = jnp.zeros((LANES,), F32)
        nslice = pl.ds(sid * NPT, NPT)
        ebase = sid * EPT

        def zrow(i, _):
            for j in range(2 * NVEC):
                zbuf[i, pl.ds(j * LANES, LANES)] = zero
            return 0

        def one_pass(q):
            db_t, e_t = db_ts[q], e_ts[q]
            qo = q * Q
            acc_o = outs[q]
            if write_enew:
                en_o = outs[NQ]
                st_o = outs[NQ + 1 + q]
            else:
                en_o = st_o = None
            lax.fori_loop(0, NPT, zrow, 0)
            pltpu.sync_copy(zbuf, acc_sh.at[nslice])
            plsc.subcore_barrier()

            def compute(b, cr):
                def row4(i4, c2):
                    out = list(c2)
                    for u in range(4):
                        i = i4 * 4 + u
                        for j in range(NVEC):
                            lsl = pl.ds(j * LANES, LANES)
                            env = (db[b][i, lsl] + ef[b][i, lsl]
                                   + cv[b][i, lsl])
                            s = 1.0 / (1.0 + jnp.exp(-env))
                            sv[b][i, pl.ds(Q + j * LANES, LANES)] = s
                            sv[b][i, lsl] = s * db[b][i, pl.ds(Q + j * LANES,
                                                               LANES)]
                            if write_enew:
                                en[b][i, lsl] = env
                                out[j] = out[j] + env
                                out[NVEC + j] = out[NVEC + j] + env * env
                    return tuple(out)

                return lax.fori_loop(0, EC // 4, row4, cr)

            def step(t, carry):
                di = []
                for b in range(2):
                    esl = pl.ds(ebase + (t * 2 + b) * EC, EC)
                    di.append([
                        pltpu.async_copy(src_hbm.at[esl], idx_s[b],
                                         sem_g[b]),
                        pltpu.async_copy(dst_hbm.at[esl], idx_d[b],
                                         sem_g[b]),
                    ])
                dg = []
                for b in range(2):
                    for d in di[b]:
                        d.wait()
                    esl = pl.ds(ebase + (t * 2 + b) * EC, EC)
                    dg.append([
                        pltpu.async_copy(db_t.at[idx_s[b]], db[b],
                                         sem_g[b]),
                        pltpu.async_copy(e_t.at[idx_d[b]], ef[b],
                                         sem_g[b]),
                        pltpu.async_copy(ce_hbm.at[esl, pl.ds(qo, Q)],
                                         cv[b], sem_g[b]),
                    ])
                prev = None
                for b in range(2):
                    for d in dg[b]:
                        d.wait()
                    carry = compute(b, carry)
                    esl = pl.ds(ebase + (t * 2 + b) * EC, EC)
                    ds = []
                    if write_enew:
                        ds.append(pltpu.async_copy(
                            en[b], en_o.at[esl, pl.ds(qo, Q)], sem_s[b]))
                    pltpu.sync_copy(sv[b], acc_sh.at[idx_d[b]], add=True)
                    if prev is not None:
                        for d in prev:
                            d.wait()
                    prev = ds
                for d in prev:
                    d.wait()
                return carry

            carry = lax.fori_loop(0, NCHUNK // 2, step,
                                  (zero,) * (2 * NVEC))
            if write_enew:
                for j in range(NVEC):
                    lsl = pl.ds(j * LANES, LANES)
                    stats_v[0, lsl] = carry[j]
                    stats_v[1, lsl] = carry[NVEC + j]
                pltpu.sync_copy(stats_v, stats_sh.at[pl.ds(sid * 8, 8)])
            plsc.subcore_barrier()
            pltpu.sync_copy(acc_sh.at[nslice], zbuf)
            pltpu.sync_copy(zbuf, acc_o.at[nslice])
            if write_enew:
                @pl.when(sid == 0)
                def _():
                    pltpu.sync_copy(stats_sh, stats_all)

                    def trow(t, cr):
                        out = list(cr)
                        for j in range(NVEC):
                            lsl = pl.ds(j * LANES, LANES)
                            out[j] = out[j] + stats_all[8 * t, lsl]
                            out[NVEC + j] = (out[NVEC + j]
                                             + stats_all[8 * t + 1, lsl])
                        return tuple(out)

                    tot = lax.fori_loop(0, NS, trow, (zero,) * (2 * NVEC))
                    for j in range(NVEC):
                        lsl = pl.ds(j * LANES, LANES)
                        stats_v[0, lsl] = tot[j]
                        stats_v[1, lsl] = tot[NVEC + j]
                    pltpu.sync_copy(stats_v, st_o)
            plsc.subcore_barrier()

        @pl.when(cid == 0)
        def _():
            one_pass(0)
            one_pass(1)

        @pl.when(cid == 1)
        def _():
            one_pass(2)
            one_pass(3)

    return pl.kernel(body, out_type=tuple(out_type), mesh=mesh,
                     scratch_types=scratch,
                     compiler_params=pltpu.CompilerParams(
                         use_tc_tiling_on_sc=False))


_R1 = 2000
_R2 = 2000


def _k1(first_layer):
    def body(*refs):
        if first_layer:
            (hidx, emb, wa, ba, wb, bb, wd, bd, we, be,
             hv_o, ah_o, *tabs) = refs
            iot = lax.broadcasted_iota(jnp.int32, (_R1, IN_DIM), 1)
            onehot = (iot == hidx[...]).astype(F32)
            hv = jnp.dot(onehot, emb[...], preferred_element_type=F32)
            hv_o[...] = hv
        else:
            (hvr, wa, ba, wb, bb, wd, bd, we, be, ah_o, *tabs) = refs
            hv = hvr[...]
        ah_o[...] = jnp.dot(hv, wa[...], preferred_element_type=F32) + ba[...]
        B = jnp.dot(hv, wb[...], preferred_element_type=F32) + bb[...]
        D = jnp.dot(hv, wd[...], preferred_element_type=F32) + bd[...]
        Ez = jnp.dot(hv, we[...], preferred_element_type=F32) + be[...]
        for q in range(NQ):
            sl = slice(q * Q, (q + 1) * Q)
            tabs[q][...] = jnp.concatenate([D[:, sl], B[:, sl]], axis=1)
            tabs[NQ + q][...] = Ez[:, sl]

    grid = (N // _R1,)
    row = lambda r: (r, 0)
    fix = lambda r: (0, 0)
    w_spec = pl.BlockSpec((HID, HID), fix)
    b_spec = pl.BlockSpec((1, HID), fix)
    full_spec = pl.BlockSpec((_R1, HID), row)
    if first_layer:
        in_specs = [pl.BlockSpec((_R1, 1), row), pl.BlockSpec((IN_DIM, HID), fix)]
    else:
        in_specs = [full_spec]
    in_specs += [w_spec, b_spec] * 4
    out_shape = [jax.ShapeDtypeStruct((N, HID), F32)] * (2 if first_layer else 1)
    out_specs = [full_spec] * (2 if first_layer else 1)
    out_shape += [jax.ShapeDtypeStruct((N, 2 * Q), F32)] * NQ
    out_specs += [pl.BlockSpec((_R1, 2 * Q), row)] * NQ
    out_shape += [jax.ShapeDtypeStruct((N, Q), F32)] * NQ
    out_specs += [pl.BlockSpec((_R1, Q), row)] * NQ
    return pl.pallas_call(body, grid=grid, in_specs=in_specs,
                          out_specs=out_specs, out_shape=out_shape)


def _k3(layer):
    def body(*refs):
        ah, nm_r, dn_r, hin, g, b = refs[:6]
        rest = refs[6:]
        hn = ah[...] + nm_r[...] / (dn_r[...] + 1e-6)
        mu = jnp.mean(hn, axis=0, keepdims=True)
        var = jnp.mean(jnp.square(hn - mu), axis=0, keepdims=True)
        hn = (hn - mu) * lax.rsqrt(var + 1e-5) * g[...] + b[...]
        ho = hin[...] + jnp.maximum(hn, 0.0)
        if layer == 3:
            w1, b1, w2, b2, w3, b3, y_o = rest
            y = jnp.maximum(jnp.dot(ho, w1[...], preferred_element_type=F32) + b1[...], 0.0)
            y = jnp.maximum(jnp.dot(y, w2[...], preferred_element_type=F32) + b2[...], 0.0)
            y_o[...] = jnp.dot(y, w3[...], preferred_element_type=F32) + b3[...]
        elif layer == 1:
            sw, sb, h_o, s_o = rest
            h_o[...] = ho
            z = jnp.dot(ho, sw[...], preferred_element_type=F32) + sb[...]
            z = z - jnp.max(z, axis=-1, keepdims=True)
            ez = jnp.exp(z)
            s_o[...] = ez / jnp.sum(ez, axis=-1, keepdims=True)
        else:
            rest[0][...] = ho

    if layer == 3:
        out_shape = jax.ShapeDtypeStruct((N, N_CLASSES), F32)
    elif layer == 1:
        out_shape = [jax.ShapeDtypeStruct((N, HID), F32),
                     jax.ShapeDtypeStruct((N, ASSIGN_DIM), F32)]
    else:
        out_shape = jax.ShapeDtypeStruct((N, HID), F32)
    return pl.pallas_call(body, out_shape=out_shape)


def _ce0():
    def body(er, ew, eb, cw, cb, cout):
        u = jnp.dot(ew[...], cw[...], preferred_element_type=F32)
        v = jnp.dot(eb[...], cw[...], preferred_element_type=F32) + cb[...]
        cout[...] = er[...] * u + v

    grid = (E // _R2,)
    row = lambda r: (r, 0)
    fix = lambda r: (0, 0)
    return pl.pallas_call(
        body, grid=grid,
        in_specs=[pl.BlockSpec((_R2, 1), row), pl.BlockSpec((1, HID), fix),
                  pl.BlockSpec((1, HID), fix), pl.BlockSpec((HID, HID), fix),
                  pl.BlockSpec((1, HID), fix)],
        out_specs=pl.BlockSpec((_R2, HID), row),
        out_shape=jax.ShapeDtypeStruct((E, HID), F32))


def _k2(mode):
    def body(*refs):
        enr = refs[0]
        if mode == 0:
            er, ew, eb, mu, inv, g, b, cw, cb = refs[1:10]
            rest = refs[10:]
            e_in = er[...] * ew[...] + eb[...]
        else:
            evp, mu, inv, g, b, cw, cb = refs[1:8]
            rest = refs[8:]
            e_in = evp[...]
        en = (enr[...] - mu[...]) * inv[...] * g[...] + b[...]
        ev = e_in + jnp.maximum(en, 0.0)
        ce = jnp.dot(ev, cw[...], preferred_element_type=F32) + cb[...]
        if mode < 2:
            ev_o, ce_o = rest
            ev_o[...] = ev
        else:
            (ce_o,) = rest
        ce_o[...] = ce

    grid = (E // _R2,)
    row = lambda r: (r, 0)
    fix = lambda r: (0, 0)
    full_spec = pl.BlockSpec((_R2, HID), row)
    p_spec = pl.BlockSpec((1, HID), fix)
    in_specs = [full_spec]
    if mode == 0:
        in_specs += [pl.BlockSpec((_R2, 1), row), p_spec, p_spec]
    else:
        in_specs += [full_spec]
    in_specs += [p_spec, p_spec, p_spec, p_spec,
                 pl.BlockSpec((HID, HID), fix), p_spec]
    out_shape = []
    out_specs = []
    if mode < 2:
        out_shape.append(jax.ShapeDtypeStruct((E, HID), F32))
        out_specs.append(full_spec)
    out_shape += [jax.ShapeDtypeStruct((E, HID), F32)]
    out_specs += [full_spec]
    return pl.pallas_call(body, grid=grid, in_specs=in_specs,
                          out_specs=out_specs, out_shape=out_shape)


def kernel(h, e, edge_index, params):
    src = edge_index[0].astype(jnp.int32)
    dst = edge_index[1].astype(jnp.int32)
    p = params
    layers = p['layers']
    r2 = lambda x: x.reshape(1, -1)
    h2 = h.astype(jnp.int32).reshape(N, 1)
    e = e.astype(F32)
    ew, eb = r2(p['emb_e_w']), r2(p['emb_e_b'])

    sc_w = _sc_gate(True)
    sc_n = _sc_gate(False)

    ce = _ce0()(e, ew, eb, layers[0]['C_w'], r2(layers[0]['C_b']))
    hv = None
    ev_prev = None
    s_out = None
    y_out = None
    for l in range(4):
        lp = layers[l]
        proj_args = (lp['A_w'], r2(lp['A_b']), lp['B_w'], r2(lp['B_b']),
                     lp['D_w'], r2(lp['D_b']), lp['E_w'], r2(lp['E_b']))
        if l == 0:
            hv, ah, *tabs = _k1(True)(h2, p['emb_h'], *proj_args)
        else:
            ah, *tabs = _k1(False)(hv, *proj_args)
        hin = hv
        if l < 3:
            sc_outs = sc_w(src, dst, *tabs, ce)
            accs = sc_outs[:NQ]
            enew = sc_outs[NQ]
            sts = sc_outs[NQ + 1:]
        else:
            accs = sc_n(src, dst, *tabs, ce)
        nums = [a[:, :Q] for a in accs]
        dens = [a[:, Q:] for a in accs]
        g, b = r2(lp['bn_h_g']), r2(lp['bn_h_b'])
        nm = jnp.concatenate(nums, axis=1)
        dn = jnp.concatenate(dens, axis=1)
        if l == 3:
            mlp = p['mlp']
            y_out = _k3(3)(ah, nm, dn, hin, g, b,
                           mlp[0]['w'], r2(mlp[0]['b']),
                           mlp[1]['w'], r2(mlp[1]['b']),
                           mlp[2]['w'], r2(mlp[2]['b']))
        elif l == 1:
            hv, s_out = _k3(1)(ah, nm, dn, hin, g, b,
                               lp['S_w'], r2(lp['S_b']))
        else:
            hv = _k3(0)(ah, nm, dn, hin, g, b)
        if l < 3:
            ssum = jnp.concatenate([s[0:1, :] for s in sts], axis=1)
            ssq = jnp.concatenate([s[1:2, :] for s in sts], axis=1)
            mu = ssum / E
            inv = lax.rsqrt(jnp.maximum(ssq / E - mu * mu, 0.0) + 1e-5)
            ge, be_ = r2(lp['bn_e_g']), r2(lp['bn_e_b'])
            cn = layers[l + 1]
            cw, cb = cn['C_w'], r2(cn['C_b'])
            if l == 0:
                ev_prev, ce = _k2(0)(enew, e, ew, eb, mu, inv, ge, be_,
                                     cw, cb)
            elif l == 1:
                ev_prev, ce = _k2(1)(enew, ev_prev, mu, inv, ge, be_,
                                     cw, cb)
            else:
                (ce,) = _k2(2)(enew, ev_prev, mu, inv, ge, be_, cw, cb)
    return (y_out, s_out)


# speedup vs baseline: 1.3708x; 1.0307x over previous
import jax
import jax.numpy as jnp
from jax import lax
from jax.experimental import pallas as pl
from jax.experimental.pallas import tpu as pltpu
from jax.experimental.pallas import tpu_sc as plsc

N = 10000
E = 320000
IN_DIM = 32
HID = 128
N_CLASSES = 6
ASSIGN_DIM = 100
F32 = jnp.float32

NS = 16
LANES = 16
Q = 32
NQ = HID // Q
EC = 80
EPT = E // NS
NCHUNK = EPT // EC
NBUF = 3
NSTEP = NCHUNK // NBUF
NTAIL = NCHUNK - NSTEP * NBUF
ZR = 125
NPT = N // NS
NVEC = Q // LANES


def _sc_gate(write_enew):
    mesh = plsc.VectorSubcoreMesh(core_axis_name="c", subcore_axis_name="s",
                                  num_cores=2, num_subcores=NS)
    out_type = [jax.ShapeDtypeStruct((N, 2 * Q), F32)] * NQ
    if write_enew:
        out_type += [jax.ShapeDtypeStruct((E, HID), F32)]
        out_type += [jax.ShapeDtypeStruct((8, Q), F32)] * NQ
    scratch = (
        [pltpu.VMEM((EC,), jnp.int32)] * NBUF
        + [pltpu.VMEM((EC,), jnp.int32)] * NBUF
        + [pltpu.VMEM((EC, 2 * Q), F32)] * NBUF
        + [pltpu.VMEM((EC, Q), F32)] * NBUF
        + [pltpu.VMEM((EC, Q), F32)] * NBUF
        + [pltpu.VMEM((EC, Q), F32)] * NBUF
        + [pltpu.VMEM((EC, 2 * Q), F32)] * NBUF
        + [
            pltpu.VMEM((8, Q), F32),
            pltpu.VMEM((8 * NS, Q), F32),
            pltpu.VMEM((ZR, 2 * Q), F32),
            pltpu.VMEM_SHARED((N, 2 * Q), F32),
            pltpu.VMEM_SHARED((8 * NS, Q), F32),
        ]
        + [pltpu.SemaphoreType.DMA] * NBUF
        + [pltpu.SemaphoreType.DMA] * NBUF
    )

    def body(src_hbm, dst_hbm, db0, db1, db2, db3, e0, e1, e2, e3,
             ce_hbm, *rest):
        n_out = (2 * NQ + 1) if write_enew else NQ
        outs = rest[:n_out]
        scr = rest[n_out:]
        idx_s = scr[0 * NBUF:1 * NBUF]
        idx_d = scr[1 * NBUF:2 * NBUF]
        db = scr[2 * NBUF:3 * NBUF]
        ef = scr[3 * NBUF:4 * NBUF]
        cv = scr[4 * NBUF:5 * NBUF]
        en = scr[5 * NBUF:6 * NBUF]
        sv = scr[6 * NBUF:7 * NBUF]
        stats_v, stats_all, zbuf, acc_sh, stats_sh = scr[7 * NBUF:
                                                         7 * NBUF + 5]
        sem_g = scr[7 * NBUF + 5:8 * NBUF + 5]
        sem_s = scr[8 * NBUF + 5:9 * NBUF + 5]
        db_ts = (db0, db1, db2, db3)
        e_ts = (e0, e1, e2, e3)
        cid = lax.axis_index("c")
        sid = lax.axis_index("s")
        zero = jnp.zeros((LANES,), F32)
        nslice = pl.ds(sid * NPT, NPT)
        ebase = sid * EPT

        def zrow(i, _):
            for j in range(2 * NVEC):
                zbuf[i, pl.ds(j * LANES, LANES)] = zero
            return 0

        def one_pass(q):
            db_t, e_t = db_ts[q], e_ts[q]
            qo = q * Q
            acc_o = outs[q]
            if write_enew:
                en_o = outs[NQ]
                st_o = outs[NQ + 1 + q]
            else:
                en_o = st_o = None
            lax.fori_loop(0, ZR, zrow, 0)
            for s in range(NPT // ZR):
                pltpu.sync_copy(zbuf,
                                acc_sh.at[pl.ds(sid * NPT + s * ZR, ZR)])
            plsc.subcore_barrier()

            def compute(b, cr):
                def row4(i4, c2):
                    out = list(c2)
                    for u in range(4):
                        i = i4 * 4 + u
                        for j in range(NVEC):
                            lsl = pl.ds(j * LANES, LANES)
                            env = (db[b][i, lsl] + ef[b][i, lsl]
                                   + cv[b][i, lsl])
                            s = 1.0 / (1.0 + jnp.exp(-env))
                            sv[b][i, pl.ds(Q + j * LANES, LANES)] = s
                            sv[b][i, lsl] = s * db[b][i, pl.ds(Q + j * LANES,
                                                               LANES)]
                            if write_enew:
                                en[b][i, lsl] = env
                                out[j] = out[j] + env
                                out[NVEC + j] = out[NVEC + j] + env * env
                    return tuple(out)

                return lax.fori_loop(0, EC // 4, row4, cr)

            def run_block(c0, carry, nb):
                di = []
                for b in range(nb):
                    esl = pl.ds(ebase + (c0 + b) * EC, EC)
                    di.append([
                        pltpu.async_copy(src_hbm.at[esl], idx_s[b],
                                         sem_g[b]),
                        pltpu.async_copy(dst_hbm.at[esl], idx_d[b],
                                         sem_g[b]),
                    ])
                dg = []
                for b in range(nb):
                    for d in di[b]:
                        d.wait()
                    esl = pl.ds(ebase + (c0 + b) * EC, EC)
                    dg.append([
                        pltpu.async_copy(db_t.at[idx_s[b]], db[b],
                                         sem_g[b]),
                        pltpu.async_copy(e_t.at[idx_d[b]], ef[b],
                                         sem_g[b]),
                        pltpu.async_copy(ce_hbm.at[esl, pl.ds(qo, Q)],
                                         cv[b], sem_g[b]),
                    ])
                prev = None
                for b in range(nb):
                    for d in dg[b]:
                        d.wait()
                    carry = compute(b, carry)
                    esl = pl.ds(ebase + (c0 + b) * EC, EC)
                    ds = []
                    if write_enew:
                        ds.append(pltpu.async_copy(
                            en[b], en_o.at[esl, pl.ds(qo, Q)], sem_s[b]))
                    pltpu.sync_copy(sv[b], acc_sh.at[idx_d[b]], add=True)
                    if prev is not None:
                        for d in prev:
                            d.wait()
                    prev = ds
                for d in prev:
                    d.wait()
                return carry

            carry = lax.fori_loop(
                0, NSTEP,
                lambda t, cr: run_block(t * NBUF, cr, NBUF),
                (zero,) * (2 * NVEC))
            if NTAIL:
                carry = run_block(NSTEP * NBUF, carry, NTAIL)
            if write_enew:
                for j in range(NVEC):
                    lsl = pl.ds(j * LANES, LANES)
                    stats_v[0, lsl] = carry[j]
                    stats_v[1, lsl] = carry[NVEC + j]
                pltpu.sync_copy(stats_v, stats_sh.at[pl.ds(sid * 8, 8)])
            plsc.subcore_barrier()
            for s in range(NPT // ZR):
                ssl = pl.ds(sid * NPT + s * ZR, ZR)
                pltpu.sync_copy(acc_sh.at[ssl], zbuf)
                pltpu.sync_copy(zbuf, acc_o.at[ssl])
            if write_enew:
                @pl.when(sid == 0)
                def _():
                    pltpu.sync_copy(stats_sh, stats_all)

                    def trow(t, cr):
                        out = list(cr)
                        for j in range(NVEC):
                            lsl = pl.ds(j * LANES, LANES)
                            out[j] = out[j] + stats_all[8 * t, lsl]
                            out[NVEC + j] = (out[NVEC + j]
                                             + stats_all[8 * t + 1, lsl])
                        return tuple(out)

                    tot = lax.fori_loop(0, NS, trow, (zero,) * (2 * NVEC))
                    for j in range(NVEC):
                        lsl = pl.ds(j * LANES, LANES)
                        stats_v[0, lsl] = tot[j]
                        stats_v[1, lsl] = tot[NVEC + j]
                    pltpu.sync_copy(stats_v, st_o)
            plsc.subcore_barrier()

        @pl.when(cid == 0)
        def _():
            one_pass(0)
            one_pass(1)

        @pl.when(cid == 1)
        def _():
            one_pass(2)
            one_pass(3)

    return pl.kernel(body, out_type=tuple(out_type), mesh=mesh,
                     scratch_types=scratch,
                     compiler_params=pltpu.CompilerParams(
                         use_tc_tiling_on_sc=False))


_R1 = 2000
_R2 = 2000


def _k1(first_layer):
    def body(*refs):
        if first_layer:
            (hidx, emb, wa, ba, wb, bb, wd, bd, we, be,
             hv_o, ah_o, *tabs) = refs
            iot = lax.broadcasted_iota(jnp.int32, (_R1, IN_DIM), 1)
            onehot = (iot == hidx[...]).astype(F32)
            hv = jnp.dot(onehot, emb[...], preferred_element_type=F32)
            hv_o[...] = hv
        else:
            (hvr, wa, ba, wb, bb, wd, bd, we, be, ah_o, *tabs) = refs
            hv = hvr[...]
        ah_o[...] = jnp.dot(hv, wa[...], preferred_element_type=F32) + ba[...]
        B = jnp.dot(hv, wb[...], preferred_element_type=F32) + bb[...]
        D = jnp.dot(hv, wd[...], preferred_element_type=F32) + bd[...]
        Ez = jnp.dot(hv, we[...], preferred_element_type=F32) + be[...]
        for q in range(NQ):
            sl = slice(q * Q, (q + 1) * Q)
            tabs[q][...] = jnp.concatenate([D[:, sl], B[:, sl]], axis=1)
            tabs[NQ + q][...] = Ez[:, sl]

    grid = (N // _R1,)
    row = lambda r: (r, 0)
    fix = lambda r: (0, 0)
    w_spec = pl.BlockSpec((HID, HID), fix)
    b_spec = pl.BlockSpec((1, HID), fix)
    full_spec = pl.BlockSpec((_R1, HID), row)
    if first_layer:
        in_specs = [pl.BlockSpec((_R1, 1), row), pl.BlockSpec((IN_DIM, HID), fix)]
    else:
        in_specs = [full_spec]
    in_specs += [w_spec, b_spec] * 4
    out_shape = [jax.ShapeDtypeStruct((N, HID), F32)] * (2 if first_layer else 1)
    out_specs = [full_spec] * (2 if first_layer else 1)
    out_shape += [jax.ShapeDtypeStruct((N, 2 * Q), F32)] * NQ
    out_specs += [pl.BlockSpec((_R1, 2 * Q), row)] * NQ
    out_shape += [jax.ShapeDtypeStruct((N, Q), F32)] * NQ
    out_specs += [pl.BlockSpec((_R1, Q), row)] * NQ
    return pl.pallas_call(body, grid=grid, in_specs=in_specs,
                          out_specs=out_specs, out_shape=out_shape)


def _k3(layer):
    def body(*refs):
        ah, nm_r, dn_r, hin, g, b = refs[:6]
        rest = refs[6:]
        hn = ah[...] + nm_r[...] / (dn_r[...] + 1e-6)
        mu = jnp.mean(hn, axis=0, keepdims=True)
        var = jnp.mean(jnp.square(hn - mu), axis=0, keepdims=True)
        hn = (hn - mu) * lax.rsqrt(var + 1e-5) * g[...] + b[...]
        ho = hin[...] + jnp.maximum(hn, 0.0)
        if layer == 3:
            w1, b1, w2, b2, w3, b3, y_o = rest
            y = jnp.maximum(jnp.dot(ho, w1[...], preferred_element_type=F32) + b1[...], 0.0)
            y = jnp.maximum(jnp.dot(y, w2[...], preferred_element_type=F32) + b2[...], 0.0)
            y_o[...] = jnp.dot(y, w3[...], preferred_element_type=F32) + b3[...]
        elif layer == 1:
            sw, sb, h_o, s_o = rest
            h_o[...] = ho
            z = jnp.dot(ho, sw[...], preferred_element_type=F32) + sb[...]
            z = z - jnp.max(z, axis=-1, keepdims=True)
            ez = jnp.exp(z)
            s_o[...] = ez / jnp.sum(ez, axis=-1, keepdims=True)
        else:
            rest[0][...] = ho

    if layer == 3:
        out_shape = jax.ShapeDtypeStruct((N, N_CLASSES), F32)
    elif layer == 1:
        out_shape = [jax.ShapeDtypeStruct((N, HID), F32),
                     jax.ShapeDtypeStruct((N, ASSIGN_DIM), F32)]
    else:
        out_shape = jax.ShapeDtypeStruct((N, HID), F32)
    return pl.pallas_call(body, out_shape=out_shape)


def _ce0():
    def body(er, ew, eb, cw, cb, cout):
        u = jnp.dot(ew[...], cw[...], preferred_element_type=F32)
        v = jnp.dot(eb[...], cw[...], preferred_element_type=F32) + cb[...]
        cout[...] = er[...] * u + v

    grid = (E // _R2,)
    row = lambda r: (r, 0)
    fix = lambda r: (0, 0)
    return pl.pallas_call(
        body, grid=grid,
        in_specs=[pl.BlockSpec((_R2, 1), row), pl.BlockSpec((1, HID), fix),
                  pl.BlockSpec((1, HID), fix), pl.BlockSpec((HID, HID), fix),
                  pl.BlockSpec((1, HID), fix)],
        out_specs=pl.BlockSpec((_R2, HID), row),
        out_shape=jax.ShapeDtypeStruct((E, HID), F32))


def _k2(mode):
    def body(*refs):
        enr = refs[0]
        if mode == 0:
            er, ew, eb, mu, inv, g, b, cw, cb = refs[1:10]
            rest = refs[10:]
            e_in = er[...] * ew[...] + eb[...]
        else:
            evp, mu, inv, g, b, cw, cb = refs[1:8]
            rest = refs[8:]
            e_in = evp[...]
        en = (enr[...] - mu[...]) * inv[...] * g[...] + b[...]
        ev = e_in + jnp.maximum(en, 0.0)
        ce = jnp.dot(ev, cw[...], preferred_element_type=F32) + cb[...]
        if mode < 2:
            ev_o, ce_o = rest
            ev_o[...] = ev
        else:
            (ce_o,) = rest
        ce_o[...] = ce

    grid = (E // _R2,)
    row = lambda r: (r, 0)
    fix = lambda r: (0, 0)
    full_spec = pl.BlockSpec((_R2, HID), row)
    p_spec = pl.BlockSpec((1, HID), fix)
    in_specs = [full_spec]
    if mode == 0:
        in_specs += [pl.BlockSpec((_R2, 1), row), p_spec, p_spec]
    else:
        in_specs += [full_spec]
    in_specs += [p_spec, p_spec, p_spec, p_spec,
                 pl.BlockSpec((HID, HID), fix), p_spec]
    out_shape = []
    out_specs = []
    if mode < 2:
        out_shape.append(jax.ShapeDtypeStruct((E, HID), F32))
        out_specs.append(full_spec)
    out_shape += [jax.ShapeDtypeStruct((E, HID), F32)]
    out_specs += [full_spec]
    return pl.pallas_call(body, grid=grid, in_specs=in_specs,
                          out_specs=out_specs, out_shape=out_shape)


def kernel(h, e, edge_index, params):
    src = edge_index[0].astype(jnp.int32)
    dst = edge_index[1].astype(jnp.int32)
    p = params
    layers = p['layers']
    r2 = lambda x: x.reshape(1, -1)
    h2 = h.astype(jnp.int32).reshape(N, 1)
    e = e.astype(F32)
    ew, eb = r2(p['emb_e_w']), r2(p['emb_e_b'])

    sc_w = _sc_gate(True)
    sc_n = _sc_gate(False)

    ce = _ce0()(e, ew, eb, layers[0]['C_w'], r2(layers[0]['C_b']))
    hv = None
    ev_prev = None
    s_out = None
    y_out = None
    for l in range(4):
        lp = layers[l]
        proj_args = (lp['A_w'], r2(lp['A_b']), lp['B_w'], r2(lp['B_b']),
                     lp['D_w'], r2(lp['D_b']), lp['E_w'], r2(lp['E_b']))
        if l == 0:
            hv, ah, *tabs = _k1(True)(h2, p['emb_h'], *proj_args)
        else:
            ah, *tabs = _k1(False)(hv, *proj_args)
        hin = hv
        if l < 3:
            sc_outs = sc_w(src, dst, *tabs, ce)
            accs = sc_outs[:NQ]
            enew = sc_outs[NQ]
            sts = sc_outs[NQ + 1:]
        else:
            accs = sc_n(src, dst, *tabs, ce)
        nums = [a[:, :Q] for a in accs]
        dens = [a[:, Q:] for a in accs]
        g, b = r2(lp['bn_h_g']), r2(lp['bn_h_b'])
        nm = jnp.concatenate(nums, axis=1)
        dn = jnp.concatenate(dens, axis=1)
        if l == 3:
            mlp = p['mlp']
            y_out = _k3(3)(ah, nm, dn, hin, g, b,
                           mlp[0]['w'], r2(mlp[0]['b']),
                           mlp[1]['w'], r2(mlp[1]['b']),
                           mlp[2]['w'], r2(mlp[2]['b']))
        elif l == 1:
            hv, s_out = _k3(1)(ah, nm, dn, hin, g, b,
                               lp['S_w'], r2(lp['S_b']))
        else:
            hv = _k3(0)(ah, nm, dn, hin, g, b)
        if l < 3:
            ssum = jnp.concatenate([s[0:1, :] for s in sts], axis=1)
            ssq = jnp.concatenate([s[1:2, :] for s in sts], axis=1)
            mu = ssum / E
            inv = lax.rsqrt(jnp.maximum(ssq / E - mu * mu, 0.0) + 1e-5)
            ge, be_ = r2(lp['bn_e_g']), r2(lp['bn_e_b'])
            cn = layers[l + 1]
            cw, cb = cn['C_w'], r2(cn['C_b'])
            if l == 0:
                ev_prev, ce = _k2(0)(enew, e, ew, eb, mu, inv, ge, be_,
                                     cw, cb)
            elif l == 1:
                ev_prev, ce = _k2(1)(enew, ev_prev, mu, inv, ge, be_,
                                     cw, cb)
            else:
                (ce,) = _k2(2)(enew, ev_prev, mu, inv, ge, be_, cw, cb)
    return (y_out, s_out)
